# baseline (speedup 1.0000x reference)
import functools

import jax
import jax.numpy as jnp
from jax import lax
from jax.experimental import pallas as pl
from jax.experimental.pallas import tpu as pltpu
from jax.experimental.pallas import tpu_sc as plsc

B = 16384
K = 16

_info = plsc.get_sparse_core_info()
NC = _info.num_cores
NS = _info.num_subcores
NW = NC * NS
BPW = B // NW
CH = 128
NCH = BPW // CH


def _gather_sc(uidx, iidx, w_table, h_table):
    mesh = plsc.VectorSubcoreMesh(core_axis_name="c", subcore_axis_name="s")

    @functools.partial(
        pl.kernel,
        mesh=mesh,
        compiler_params=pltpu.CompilerParams(use_tc_tiling_on_sc=False),
        out_type=(
            jax.ShapeDtypeStruct((B, K), jnp.float32),
            jax.ShapeDtypeStruct((B, K), jnp.float32),
        ),
        scratch_types=[
            pltpu.VMEM((NCH, CH), jnp.int32),
            pltpu.VMEM((NCH, CH), jnp.int32),
            pltpu.VMEM((BPW, K), jnp.float32),
            pltpu.VMEM((BPW, K), jnp.float32),
            pltpu.SemaphoreType.DMA,
        ],
    )
    def k(uidx_hbm, iidx_hbm, w_hbm, h_hbm, u_out, v_out,
          uidx_v, iidx_v, urows, vrows, sem):
        wid = lax.axis_index("s") * NC + lax.axis_index("c")
        pltpu.sync_copy(uidx_hbm.at[wid], uidx_v)
        pltpu.sync_copy(iidx_hbm.at[wid], iidx_v)
        copies = []
        for j in range(NCH):
            copies.append(pltpu.async_copy(
                w_hbm.at[uidx_v.at[j]], urows.at[pl.ds(j * CH, CH)], sem))
            copies.append(pltpu.async_copy(
                h_hbm.at[iidx_v.at[j]], vrows.at[pl.ds(j * CH, CH)], sem))
        for c in copies:
            c.wait()
        base = wid * BPW
        pltpu.sync_copy(urows, u_out.at[pl.ds(base, BPW)])
        pltpu.sync_copy(vrows, v_out.at[pl.ds(base, BPW)])

    return k(uidx, iidx, w_table, h_table)


def _mlp_body(u_ref, v_ref, a_ref, bt_ref, b1_ref, w2_ref, o_ref):
    h = jnp.dot(u_ref[...], a_ref[...], preferred_element_type=jnp.float32)
    h = h + jnp.dot(v_ref[...], bt_ref[...], preferred_element_type=jnp.float32)
    h = jnp.maximum(h + b1_ref[...], 0.0)
    o_ref[...] = jnp.dot(h, w2_ref[...], preferred_element_type=jnp.float32)


def _mlp_tc(u, v, w1at, w1bt, b1r, w2t):
    return pl.pallas_call(
        _mlp_body,
        out_shape=jax.ShapeDtypeStruct((B, 1), jnp.float32),
    )(u, v, w1at, w1bt, b1r, w2t)


def kernel(x, W_table, H_table, W_r_table, H_r_table, W1, b1, W2):
    uidx = x[:, 0].reshape(NW, NCH, CH)
    iidx = x[:, 1].reshape(NW, NCH, CH)
    u, v = _gather_sc(uidx, iidx, W_table, H_table)
    w1at = W1[:, :K].T
    w1bt = W1[:, K:].T
    return _mlp_tc(u, v, w1at, w1bt, b1.reshape(1, K), W2.T)


# speedup vs baseline: 1.0104x; 1.0104x over previous
import functools

import jax
import jax.numpy as jnp
from jax import lax
from jax.experimental import pallas as pl
from jax.experimental.pallas import tpu as pltpu
from jax.experimental.pallas import tpu_sc as plsc

B = 16384
K = 16
PACK = 8
WIDE = PACK * K

_info = plsc.get_sparse_core_info()
NC = _info.num_cores
NS = _info.num_subcores
NW = NC * NS
BPW = B // NW
CH = 128
NCH = BPW // CH
OROW = BPW // PACK


def _gather_sc(uwidx, uloff, iwidx, iloff, w_wide, h_wide):
    mesh = plsc.VectorSubcoreMesh(core_axis_name="c", subcore_axis_name="s")

    @functools.partial(
        pl.kernel,
        mesh=mesh,
        compiler_params=pltpu.CompilerParams(needs_layout_passes=False),
        out_type=(
            jax.ShapeDtypeStruct((B // PACK, WIDE), jnp.float32),
            jax.ShapeDtypeStruct((B // PACK, WIDE), jnp.float32),
        ),
        scratch_types=[
            pltpu.VMEM((NCH, CH), jnp.int32),
            pltpu.VMEM((NCH, CH), jnp.int32),
            pltpu.VMEM((NCH, CH), jnp.int32),
            pltpu.VMEM((NCH, CH), jnp.int32),
            pltpu.VMEM((CH, WIDE), jnp.float32),
            pltpu.VMEM((CH, WIDE), jnp.float32),
            pltpu.VMEM((CH, WIDE), jnp.float32),
            pltpu.VMEM((CH, WIDE), jnp.float32),
            pltpu.VMEM((OROW, WIDE), jnp.float32),
            pltpu.VMEM((OROW, WIDE), jnp.float32),
            pltpu.SemaphoreType.DMA,
            pltpu.SemaphoreType.DMA,
            pltpu.SemaphoreType.DMA,
            pltpu.SemaphoreType.DMA,
        ],
    )
    def k(uwidx_hbm, uloff_hbm, iwidx_hbm, iloff_hbm, w_hbm, h_hbm,
          u_out, v_out,
          uwidx_v, uloff_v, iwidx_v, iloff_v,
          ubuf0, ubuf1, vbuf0, vbuf1, upack, vpack,
          usem0, usem1, vsem0, vsem1):
        wid = lax.axis_index("s") * NC + lax.axis_index("c")
        pltpu.sync_copy(uwidx_hbm.at[pl.ds(wid * NCH, NCH)], uwidx_v)
        pltpu.sync_copy(uloff_hbm.at[pl.ds(wid * NCH, NCH)], uloff_v)
        pltpu.sync_copy(iwidx_hbm.at[pl.ds(wid * NCH, NCH)], iwidx_v)
        pltpu.sync_copy(iloff_hbm.at[pl.ds(wid * NCH, NCH)], iloff_v)

        ubufs = (ubuf0, ubuf1)
        vbufs = (vbuf0, vbuf1)
        usems = (usem0, usem1)
        vsems = (vsem0, vsem1)

        def fire(tbl, idx_v, bufs, sems, j):
            return pltpu.async_copy(tbl.at[idx_v.at[j]], bufs[j % 2],
                                    sems[j % 2])

        iota = lax.iota(jnp.int32, K)
        packrow = lax.shift_right_logical(iota, 3)
        packcol = (iota & 7) * K

        def extract(buf, loff_v, pack_buf, j):
            for g in range(CH // K):
                offv = loff_v[j, pl.ds(g * K, K)]
                src_row = iota + g * K
                dst_row = packrow + (j * (CH // PACK) + 2 * g)
                for c in range(K):
                    vals = plsc.load_gather(buf, [src_row, offv + c])
                    plsc.store_scatter(pack_buf, [dst_row, packcol + c], vals)

        cps = {}
        for j in range(2):
            cps[("u", j)] = fire(w_hbm, uwidx_v, ubufs, usems, j)
            cps[("v", j)] = fire(h_hbm, iwidx_v, vbufs, vsems, j)
        for j in range(NCH):
            cps[("u", j)].wait()
            extract(ubufs[j % 2], uloff_v, upack, j)
            if j + 2 < NCH:
                cps[("u", j + 2)] = fire(w_hbm, uwidx_v, ubufs, usems, j + 2)
            cps[("v", j)].wait()
            extract(vbufs[j % 2], iloff_v, vpack, j)
            if j + 2 < NCH:
                cps[("v", j + 2)] = fire(h_hbm, iwidx_v, vbufs, vsems, j + 2)

        pltpu.sync_copy(upack, u_out.at[pl.ds(wid * OROW, OROW)])
        pltpu.sync_copy(vpack, v_out.at[pl.ds(wid * OROW, OROW)])

    return k(uwidx, uloff, iwidx, iloff, w_wide, h_wide)


def _mlp_body(u_ref, v_ref, a_ref, b_ref, b1_ref, w2_ref, o_ref):
    h = jnp.dot(u_ref[...], a_ref[...], preferred_element_type=jnp.float32)
    h = h + jnp.dot(v_ref[...], b_ref[...], preferred_element_type=jnp.float32)
    h = jnp.maximum(h + b1_ref[...], 0.0)
    o_ref[...] = jnp.dot(h, w2_ref[...], preferred_element_type=jnp.float32)


def _mlp_tc(u, v, a, bm, b1t, w2b):
    return pl.pallas_call(
        _mlp_body,
        out_shape=jax.ShapeDtypeStruct((B // PACK, PACK), jnp.float32),
    )(u, v, a, bm, b1t, w2b)


def kernel(x, W_table, H_table, W_r_table, H_r_table, W1, b1, W2):
    uidx = x[:, 0]
    iidx = x[:, 1]
    uwidx = lax.shift_right_logical(uidx, 3).reshape(NW * NCH, CH)
    iwidx = lax.shift_right_logical(iidx, 3).reshape(NW * NCH, CH)
    uloff = ((uidx & 7) * K).reshape(NW * NCH, CH)
    iloff = ((iidx & 7) * K).reshape(NW * NCH, CH)
    w_wide = W_table.reshape(-1, WIDE)
    h_wide = H_table.reshape(-1, WIDE)
    u, v = _gather_sc(uwidx, uloff, iwidx, iloff, w_wide, h_wide)

    eye = jnp.eye(PACK, dtype=jnp.float32)
    a = jnp.kron(eye, W1[:, :K].T)
    bm = jnp.kron(eye, W1[:, K:].T)
    b1t = jnp.tile(b1, PACK).reshape(1, WIDE)
    w2b = jnp.kron(eye, W2.reshape(K, 1))
    out = _mlp_tc(u, v, a, bm, b1t, w2b)
    return out.reshape(B, 1)


# speedup vs baseline: 4.8637x; 4.8135x over previous
import functools

import jax
import jax.numpy as jnp
from jax import lax
from jax.experimental import pallas as pl
from jax.experimental.pallas import tpu as pltpu
from jax.experimental.pallas import tpu_sc as plsc

B = 16384
K = 16
PACK = 8
WIDE = PACK * K

_info = plsc.get_sparse_core_info()
NC = _info.num_cores
NS = _info.num_subcores
NW = NC * NS
BPW = B // NW
NBUF = 8
OROW = BPW // PACK


def _gather_sc(ustart, ulane, istart, ilane, wt, ht):
    mesh = plsc.VectorSubcoreMesh(core_axis_name="c", subcore_axis_name="s")

    scratch = (
        [pltpu.VMEM((BPW + K,), jnp.int32)] * 4
        + [pltpu.VMEM((K, WIDE), jnp.float32)] * (2 * NBUF)
        + [pltpu.VMEM((OROW, WIDE), jnp.float32)] * 2
        + [pltpu.SemaphoreType.DMA] * (2 * NBUF)
    )

    @functools.partial(
        pl.kernel,
        mesh=mesh,
        compiler_params=pltpu.CompilerParams(needs_layout_passes=False),
        out_type=(
            jax.ShapeDtypeStruct((B // PACK, WIDE), jnp.float32),
            jax.ShapeDtypeStruct((B // PACK, WIDE), jnp.float32),
        ),
        scratch_types=scratch,
    )
    def k(ustart_hbm, ulane_hbm, istart_hbm, ilane_hbm, w_hbm, h_hbm,
          u_out, v_out, *sc):
        ustart_v, ulane_v, istart_v, ilane_v = sc[0:4]
        ubufs = sc[4:4 + NBUF]
        vbufs = sc[4 + NBUF:4 + 2 * NBUF]
        upack, vpack = sc[4 + 2 * NBUF:6 + 2 * NBUF]
        usems = sc[6 + 2 * NBUF:6 + 3 * NBUF]
        vsems = sc[6 + 3 * NBUF:6 + 4 * NBUF]

        wid = lax.axis_index("s") * NC + lax.axis_index("c")
        base = wid * BPW
        pltpu.sync_copy(ustart_hbm.at[pl.ds(base, BPW)],
                        ustart_v.at[pl.ds(0, BPW)])
        pltpu.sync_copy(ulane_hbm.at[pl.ds(base, BPW)],
                        ulane_v.at[pl.ds(0, BPW)])
        pltpu.sync_copy(istart_hbm.at[pl.ds(base, BPW)],
                        istart_v.at[pl.ds(0, BPW)])
        pltpu.sync_copy(ilane_hbm.at[pl.ds(base, BPW)],
                        ilane_v.at[pl.ds(0, BPW)])

        iota = lax.iota(jnp.int32, K)

        def fire(tbl, start_v, buf, sem, r):
            s = pl.multiple_of(start_v[pl.ds(r, K)][0], WIDE)
            pltpu.async_copy(tbl.at[:, pl.ds(s, WIDE)], buf, sem)

        def wait(tbl, buf, sem):
            pltpu.make_async_copy(tbl.at[:, pl.ds(0, WIDE)], buf, sem).wait()

        def extract(buf, lane_v, pack_buf, r):
            col = plsc.load_gather(lane_v, [jnp.full((K,), r, jnp.int32)])
            vals = plsc.load_gather(buf, [iota, col])
            prow = jnp.full((K,), lax.shift_right_logical(r, 3), jnp.int32)
            pcol = (r & 7) * K + iota
            plsc.store_scatter(pack_buf, [prow, pcol], vals)

        for b in range(NBUF):
            fire(w_hbm, ustart_v, ubufs[b], usems[b], b)
            fire(h_hbm, istart_v, vbufs[b], vsems[b], b)

        def body(t, carry):
            for b in range(NBUF):
                r = t * NBUF + b
                wait(w_hbm, ubufs[b], usems[b])
                extract(ubufs[b], ulane_v, upack, r)
                fire(w_hbm, ustart_v, ubufs[b], usems[b], r + NBUF)
                wait(h_hbm, vbufs[b], vsems[b])
                extract(vbufs[b], ilane_v, vpack, r)
                fire(h_hbm, istart_v, vbufs[b], vsems[b], r + NBUF)
            return carry

        lax.fori_loop(0, BPW // NBUF - 1, body, 0)

        for b in range(NBUF):
            r = BPW - NBUF + b
            wait(w_hbm, ubufs[b], usems[b])
            extract(ubufs[b], ulane_v, upack, r)
            wait(h_hbm, vbufs[b], vsems[b])
            extract(vbufs[b], ilane_v, vpack, r)

        pltpu.sync_copy(upack, u_out.at[pl.ds(wid * OROW, OROW)])
        pltpu.sync_copy(vpack, v_out.at[pl.ds(wid * OROW, OROW)])

    return k(ustart, ulane, istart, ilane, wt, ht)


def _mlp_body(u_ref, v_ref, a_ref, b_ref, b1_ref, w2_ref, o_ref):
    h = jnp.dot(u_ref[...], a_ref[...], preferred_element_type=jnp.float32)
    h = h + jnp.dot(v_ref[...], b_ref[...], preferred_element_type=jnp.float32)
    h = jnp.maximum(h + b1_ref[...], 0.0)
    o_ref[...] = jnp.dot(h, w2_ref[...], preferred_element_type=jnp.float32)


def _mlp_tc(u, v, a, bm, b1t, w2b):
    return pl.pallas_call(
        _mlp_body,
        out_shape=jax.ShapeDtypeStruct((B // PACK, PACK), jnp.float32),
    )(u, v, a, bm, b1t, w2b)


def kernel(x, W_table, H_table, W_r_table, H_r_table, W1, b1, W2):
    uidx = x[:, 0]
    iidx = x[:, 1]
    ustart = lax.shift_right_logical(uidx, 7) * WIDE
    istart = lax.shift_right_logical(iidx, 7) * WIDE
    ulane = uidx & 127
    ilane = iidx & 127
    wt = W_table.T
    ht = H_table.T
    u, v = _gather_sc(ustart, ulane, istart, ilane, wt, ht)

    eye = jnp.eye(PACK, dtype=jnp.float32)
    a = jnp.kron(eye, W1[:, :K].T)
    bm = jnp.kron(eye, W1[:, K:].T)
    b1t = jnp.tile(b1, PACK).reshape(1, WIDE)
    w2b = jnp.kron(eye, W2.reshape(K, 1))
    out = _mlp_tc(u, v, a, bm, b1t, w2b)
    return out.reshape(B, 1)


# speedup vs baseline: 4.8948x; 1.0064x over previous
import functools

import jax
import jax.numpy as jnp
from jax import lax
from jax.experimental import pallas as pl
from jax.experimental.pallas import tpu as pltpu
from jax.experimental.pallas import tpu_sc as plsc

B = 16384
K = 16
PACK = 8
WIDE = PACK * K

_info = plsc.get_sparse_core_info()
NC = _info.num_cores
NS = _info.num_subcores
NW = NC * NS
BPW = B // NW
NBUF = 8
OROW = BPW // PACK


def _gather_sc(uidx, iidx, wt, ht):
    mesh = plsc.VectorSubcoreMesh(core_axis_name="c", subcore_axis_name="s")

    scratch = (
        [pltpu.VMEM((BPW + K,), jnp.int32)] * 2
        + [pltpu.VMEM((K, WIDE), jnp.float32)] * (2 * NBUF)
        + [pltpu.VMEM((OROW, WIDE), jnp.float32)] * 2
        + [pltpu.SemaphoreType.DMA] * (2 * NBUF)
    )

    @functools.partial(
        pl.kernel,
        mesh=mesh,
        compiler_params=pltpu.CompilerParams(needs_layout_passes=False),
        out_type=(
            jax.ShapeDtypeStruct((B // PACK, WIDE), jnp.float32),
            jax.ShapeDtypeStruct((B // PACK, WIDE), jnp.float32),
        ),
        scratch_types=scratch,
    )
    def k(uidx_hbm, iidx_hbm, w_hbm, h_hbm,
          u_out, v_out, *sc):
        uidx_v, iidx_v = sc[0:2]
        ubufs = sc[2:2 + NBUF]
        vbufs = sc[2 + NBUF:2 + 2 * NBUF]
        upack, vpack = sc[2 + 2 * NBUF:4 + 2 * NBUF]
        usems = sc[4 + 2 * NBUF:4 + 3 * NBUF]
        vsems = sc[4 + 3 * NBUF:4 + 4 * NBUF]

        wid = lax.axis_index("s") * NC + lax.axis_index("c")
        base = wid * BPW
        pltpu.sync_copy(uidx_hbm.at[pl.ds(base, BPW)],
                        uidx_v.at[pl.ds(0, BPW)])
        pltpu.sync_copy(iidx_hbm.at[pl.ds(base, BPW)],
                        iidx_v.at[pl.ds(0, BPW)])

        iota = lax.iota(jnp.int32, K)

        def fire(tbl, idx_v, buf, sem, r):
            raw = idx_v[pl.ds(r, K)][0]
            s = pl.multiple_of(lax.shift_right_logical(raw, 7) * WIDE, WIDE)
            pltpu.async_copy(tbl.at[:, pl.ds(s, WIDE)], buf, sem)

        def wait(tbl, buf, sem):
            pltpu.make_async_copy(tbl.at[:, pl.ds(0, WIDE)], buf, sem).wait()

        def extract(buf, idx_v, pack_buf, r):
            col = plsc.load_gather(idx_v, [jnp.full((K,), r, jnp.int32)]) & 127
            vals = plsc.load_gather(buf, [iota, col])
            prow = jnp.full((K,), lax.shift_right_logical(r, 3), jnp.int32)
            pcol = (r & 7) * K + iota
            plsc.store_scatter(pack_buf, [prow, pcol], vals)

        for b in range(NBUF):
            fire(w_hbm, uidx_v, ubufs[b], usems[b], b)
            fire(h_hbm, iidx_v, vbufs[b], vsems[b], b)

        def body(t, carry):
            for b in range(NBUF):
                r = t * NBUF + b
                wait(w_hbm, ubufs[b], usems[b])
                extract(ubufs[b], uidx_v, upack, r)
                fire(w_hbm, uidx_v, ubufs[b], usems[b], r + NBUF)
                wait(h_hbm, vbufs[b], vsems[b])
                extract(vbufs[b], iidx_v, vpack, r)
                fire(h_hbm, iidx_v, vbufs[b], vsems[b], r + NBUF)
            return carry

        lax.fori_loop(0, BPW // NBUF - 1, body, 0)

        for b in range(NBUF):
            r = BPW - NBUF + b
            wait(w_hbm, ubufs[b], usems[b])
            extract(ubufs[b], uidx_v, upack, r)
            wait(h_hbm, vbufs[b], vsems[b])
            extract(vbufs[b], iidx_v, vpack, r)

        pltpu.sync_copy(upack, u_out.at[pl.ds(wid * OROW, OROW)])
        pltpu.sync_copy(vpack, v_out.at[pl.ds(wid * OROW, OROW)])

    return k(uidx, iidx, wt, ht)


def _mlp_body(u_ref, v_ref, a_ref, b_ref, b1_ref, w2_ref, o_ref):
    h = jnp.dot(u_ref[...], a_ref[...], preferred_element_type=jnp.float32)
    h = h + jnp.dot(v_ref[...], b_ref[...], preferred_element_type=jnp.float32)
    h = jnp.maximum(h + b1_ref[...], 0.0)
    o_ref[...] = jnp.dot(h, w2_ref[...], preferred_element_type=jnp.float32)


def _mlp_tc(u, v, a, bm, b1t, w2b):
    return pl.pallas_call(
        _mlp_body,
        out_shape=jax.ShapeDtypeStruct((B // PACK, PACK), jnp.float32),
    )(u, v, a, bm, b1t, w2b)


def kernel(x, W_table, H_table, W_r_table, H_r_table, W1, b1, W2):
    uidx = x[:, 0]
    iidx = x[:, 1]
    wt = W_table.T
    ht = H_table.T
    u, v = _gather_sc(uidx, iidx, wt, ht)

    eye = jnp.eye(PACK, dtype=jnp.float32)
    a = jnp.kron(eye, W1[:, :K].T)
    bm = jnp.kron(eye, W1[:, K:].T)
    b1t = jnp.tile(b1, PACK).reshape(1, WIDE)
    w2b = jnp.kron(eye, W2.reshape(K, 1))
    out = _mlp_tc(u, v, a, bm, b1t, w2b)
    return out.reshape(B, 1)
